# baseline (speedup 1.0000x reference)
import functools

import jax
import jax.numpy as jnp
from jax import lax
from jax.experimental import pallas as pl
from jax.experimental.pallas import tpu as pltpu
from jax.experimental.pallas import tpu_sc as plsc

ALPHA = 0.2
MOM = 0.8

_PREC = lax.Precision.DEFAULT


def _normalize(x):
    n = jnp.sqrt(jnp.sum(x * x, axis=-1, keepdims=True))
    return x / jnp.maximum(n, 1e-12)


def _read_kernel(text_ref, cache_ref, w1_ref, w2_ref, out_ref, loss_ref,
                 *, halves):
    i = pl.program_id(0)
    cache = cache_ref[...]
    hb = text_ref.shape[0] // halves
    part = jnp.zeros((), jnp.float32)
    for hh in range(halves):
        t = text_ref[pl.ds(hh * hb, hb), :]
        base = _normalize(t).astype(jnp.bfloat16)
        s = lax.dot_general(base, cache, (((1,), (1,)), ((), ())),
                            preferred_element_type=jnp.float32,
                            precision=_PREC)
        e = jnp.exp(s).astype(jnp.bfloat16)
        denom = jnp.sum(e.astype(jnp.float32), axis=1, keepdims=True)
        fsum = lax.dot_general(e, cache, (((1,), (0,)), ((), ())),
                               preferred_element_type=jnp.float32,
                               precision=_PREC)
        fine = fsum / denom
        o1 = lax.dot_general(t.astype(jnp.bfloat16), w1_ref[...],
                             (((1,), (1,)), ((), ())),
                             preferred_element_type=jnp.float32,
                             precision=_PREC)
        o2 = lax.dot_general(fine.astype(jnp.bfloat16), w2_ref[...],
                             (((1,), (1,)), ((), ())),
                             preferred_element_type=jnp.float32,
                             precision=_PREC)
        out = ALPHA * (o1 + o2) + t
        out_ref[pl.ds(hh * hb, hb), :] = out
        od = _normalize(out)
        part = part + jnp.sum(jnp.abs(od - t))

    @pl.when(i == 0)
    def _():
        loss_ref[...] = jnp.zeros_like(loss_ref)

    loss_ref[...] += part.reshape(1, 1)


def _stats_kernel(img_ref, cache_ref, swrows_ref, rarg_ref, ecol_ref,
                  wsum_ref, cmax_ref, *, nb, nsteps, halves):
    i = pl.program_id(0)
    cache = cache_ref[...]
    hb = img_ref.shape[0] // halves
    pcmax = None
    cnt = None
    for hh in range(halves):
        t = img_ref[pl.ds(hh * hb, hb), :]
        n = jnp.sqrt(jnp.sum(t * t, axis=1, keepdims=True))
        n = jnp.maximum(n, 1e-12)
        bi = t / n
        s = lax.dot_general(bi.astype(jnp.bfloat16), cache,
                            (((1,), (1,)), ((), ())),
                            preferred_element_type=jnp.float32,
                            precision=_PREC)
        mm = s.shape[1]
        rmax = jnp.max(s, axis=1, keepdims=True)
        jidx = lax.broadcasted_iota(jnp.int32, s.shape, 1)
        rarg = jnp.min(jnp.where(s == rmax, jidx, mm), axis=1)
        swrows_ref[pl.ds(hh * hb, hb), :] = jnp.exp(rmax) * bi
        rarg_ref[0, pl.ds(i * nb + hh * hb, hb)] = rarg
        pm = jnp.max(s, axis=0, keepdims=True)
        ct = jnp.sum(jnp.where(rarg[:, None] == jidx, 1.0, 0.0),
                     axis=0, keepdims=True)
        pcmax = pm if pcmax is None else jnp.maximum(pcmax, pm)
        cnt = ct if cnt is None else cnt + ct

    @pl.when(i == 0)
    def _():
        cmax_ref[...] = pcmax
        wsum_ref[...] = cnt

    @pl.when(i > 0)
    def _():
        cmax_ref[...] = jnp.maximum(cmax_ref[...], pcmax)
        wsum_ref[...] += cnt

    ecol_ref[...] = jnp.exp(-cmax_ref[...])


def _sc_scatter_body(swrows_hbm, rarg_hbm, out_hbm,
                     rows, zbuf, idx_a, idx_b, shared,
                     *, m, q, tpt):
    c = lax.axis_index("c")
    sid = lax.axis_index("s")
    base = sid * tpt
    zero16 = jnp.zeros((16,), jnp.float32)

    pltpu.sync_copy(rarg_hbm.at[0, pl.ds(base, 128)], idx_a)
    pltpu.sync_copy(rarg_hbm.at[0, pl.ds(base + 128, 128)], idx_b)

    def zbody(t, carry):
        for r in range(q // 16):
            zbuf[t, pl.ds(r * 16, 16)] = zero16
        return carry

    lax.fori_loop(0, tpt, zbody, 0)

    for h in range(3):
        qi = c * 3 + h

        pltpu.sync_copy(zbuf, shared.at[pl.ds(base, tpt)])

        pltpu.sync_copy(swrows_hbm.at[pl.ds(base, tpt), pl.ds(qi * q, q)],
                        rows)

        plsc.subcore_barrier()

        pltpu.sync_copy(rows.at[pl.ds(0, 128)], shared.at[idx_a], add=True)
        pltpu.sync_copy(rows.at[pl.ds(128, 128)], shared.at[idx_b], add=True)
        plsc.subcore_barrier()

        pltpu.sync_copy(shared.at[pl.ds(base, tpt)],
                        out_hbm.at[qi, pl.ds(base, tpt)])


def _final_kernel(cache_ref, c0_ref, c1_ref, c2_ref, c3_ref, c4_ref, c5_ref,
                  ecol_ref, wsum_ref, upd_ref, *, mb, q):
    i = pl.program_id(0)
    c = cache_ref[...]
    ws = wsum_ref[0, pl.ds(i * mb, mb)]
    ec = ecol_ref[0, pl.ds(i * mb, mb)]
    upd = ws[:, None] > 0
    scale = (1.0 - MOM) * ec[:, None]
    quarters = (c0_ref, c1_ref, c2_ref, c3_ref, c4_ref, c5_ref)
    blended = []
    n2 = jnp.zeros((c.shape[0], 1), jnp.float32)
    for k, qref in enumerate(quarters):
        b = jnp.where(upd, MOM * c[:, k * q:(k + 1) * q]
                      + scale * qref[0], c[:, k * q:(k + 1) * q])
        blended.append(b)
        n2 = n2 + jnp.sum(b * b, axis=1, keepdims=True)
    inv = 1.0 / jnp.maximum(jnp.sqrt(n2), 1e-12)
    for k, b in enumerate(blended):
        upd_ref[:, k * q:(k + 1) * q] = b * inv


def kernel(text_token, image_token, W, cache):
    C, D = text_token.shape
    N = image_token.shape[0]
    M = cache.shape[0]
    CB = 512
    NB = 512
    MB = 512
    Q = D // 6
    TPT = N // 16
    W1 = W[:, :D].astype(jnp.bfloat16)
    W2 = W[:, D:].astype(jnp.bfloat16)
    cache_bf = cache.astype(jnp.bfloat16)

    nsteps = N // NB
    swrows, rarg, ecol, wsum = pl.pallas_call(
        functools.partial(_stats_kernel, nb=NB, nsteps=nsteps, halves=4),
        grid=(nsteps,),
        in_specs=[
            pl.BlockSpec((NB, D), lambda i: (i, 0)),
            pl.BlockSpec((M, D), lambda i: (0, 0)),
        ],
        out_specs=[
            pl.BlockSpec((NB, D), lambda i: (i, 0)),
            pl.BlockSpec((1, N), lambda i: (0, 0)),
            pl.BlockSpec((1, M), lambda i: (0, 0)),
            pl.BlockSpec((1, M), lambda i: (0, 0)),
        ],
        out_shape=[
            jax.ShapeDtypeStruct((N, D), jnp.float32),
            jax.ShapeDtypeStruct((1, N), jnp.int32),
            jax.ShapeDtypeStruct((1, M), jnp.float32),
            jax.ShapeDtypeStruct((1, M), jnp.float32),
        ],
        scratch_shapes=[
            pltpu.VMEM((1, M), jnp.float32),
        ],
    )(image_token, cache_bf)

    sc_scatter = functools.partial(
        pl.kernel,
        mesh=plsc.VectorSubcoreMesh(core_axis_name="c", subcore_axis_name="s"),
        out_type=jax.ShapeDtypeStruct((6, M, Q), jnp.float32),
        scratch_types=[
            pltpu.VMEM((TPT, Q), jnp.float32),
            pltpu.VMEM((TPT, Q), jnp.float32),
            pltpu.VMEM((128,), jnp.int32),
            pltpu.VMEM((128,), jnp.int32),
            pltpu.VMEM_SHARED((M, Q), jnp.float32),
        ],
    )(functools.partial(_sc_scatter_body, m=M, q=Q, tpt=TPT))
    contrib4 = sc_scatter(swrows, rarg)

    out, loss_sum = pl.pallas_call(
        functools.partial(_read_kernel, halves=4),
        grid=(C // CB,),
        in_specs=[
            pl.BlockSpec((CB, D), lambda i: (i, 0)),
            pl.BlockSpec((M, D), lambda i: (0, 0)),
            pl.BlockSpec((D, D), lambda i: (0, 0)),
            pl.BlockSpec((D, D), lambda i: (0, 0)),
        ],
        out_specs=[
            pl.BlockSpec((CB, D), lambda i: (i, 0)),
            pl.BlockSpec((1, 1), lambda i: (0, 0)),
        ],
        out_shape=[
            jax.ShapeDtypeStruct((C, D), jnp.float32),
            jax.ShapeDtypeStruct((1, 1), jnp.float32),
        ],
    )(text_token, cache_bf, W1, W2)
    loss = loss_sum[0, 0] / (C * D)

    updated = pl.pallas_call(
        functools.partial(_final_kernel, mb=MB, q=Q),
        grid=(M // MB,),
        in_specs=[
            pl.BlockSpec((MB, D), lambda i: (i, 0)),
            pl.BlockSpec((1, MB, Q), lambda i: (0, i, 0)),
            pl.BlockSpec((1, MB, Q), lambda i: (1, i, 0)),
            pl.BlockSpec((1, MB, Q), lambda i: (2, i, 0)),
            pl.BlockSpec((1, MB, Q), lambda i: (3, i, 0)),
            pl.BlockSpec((1, MB, Q), lambda i: (4, i, 0)),
            pl.BlockSpec((1, MB, Q), lambda i: (5, i, 0)),
            pl.BlockSpec((1, M), lambda i: (0, 0)),
            pl.BlockSpec((1, M), lambda i: (0, 0)),
        ],
        out_specs=pl.BlockSpec((MB, D), lambda i: (i, 0)),
        out_shape=jax.ShapeDtypeStruct((M, D), jnp.float32),
    )(cache, contrib4, contrib4, contrib4, contrib4, contrib4, contrib4,
      ecol, wsum)

    return (out, loss, updated)


# speedup vs baseline: 1.5280x; 1.5280x over previous
import functools

import jax
import jax.numpy as jnp
from jax import lax
from jax.experimental import pallas as pl
from jax.experimental.pallas import tpu as pltpu
from jax.experimental.pallas import tpu_sc as plsc

ALPHA = 0.2
MOM = 0.8

_PREC = lax.Precision.DEFAULT


def _normalize(x):
    n = jnp.sqrt(jnp.sum(x * x, axis=-1, keepdims=True))
    return x / jnp.maximum(n, 1e-12)


def _read_kernel(text_ref, cache_ref, w1_ref, w2_ref, out_ref, loss_ref,
                 *, halves):
    i = pl.program_id(0)
    cache = cache_ref[...]
    hb = text_ref.shape[0] // halves
    part = jnp.zeros((), jnp.float32)
    for hh in range(halves):
        t = text_ref[pl.ds(hh * hb, hb), :]
        base = _normalize(t).astype(jnp.bfloat16)
        s = lax.dot_general(base, cache, (((1,), (1,)), ((), ())),
                            preferred_element_type=jnp.float32,
                            precision=_PREC)
        e32 = jnp.exp(s)
        e = e32.astype(jnp.bfloat16)
        denom = jnp.sum(e32, axis=1, keepdims=True)
        fsum = lax.dot_general(e, cache, (((1,), (0,)), ((), ())),
                               preferred_element_type=jnp.float32,
                               precision=_PREC)
        fine = fsum / denom
        o1 = lax.dot_general(t.astype(jnp.bfloat16), w1_ref[...],
                             (((1,), (1,)), ((), ())),
                             preferred_element_type=jnp.float32,
                             precision=_PREC)
        o2 = lax.dot_general(fine.astype(jnp.bfloat16), w2_ref[...],
                             (((1,), (1,)), ((), ())),
                             preferred_element_type=jnp.float32,
                             precision=_PREC)
        out = ALPHA * (o1 + o2) + t
        out_ref[pl.ds(hh * hb, hb), :] = out
        od = _normalize(out)
        part = part + jnp.sum(jnp.abs(od - t))

    @pl.when(i == 0)
    def _():
        loss_ref[...] = jnp.zeros_like(loss_ref)

    loss_ref[...] += part.reshape(1, 1)


def _stats_kernel(img_ref, cache_ref, swrows_ref, rarg_ref, ecol_ref,
                  wsum_ref, cmax_ref, *, nb, nsteps, halves):
    i = pl.program_id(0)
    cache = cache_ref[...]
    hb = img_ref.shape[0] // halves
    pcmax = None
    cnt = None
    for hh in range(halves):
        t = img_ref[pl.ds(hh * hb, hb), :]
        n = jnp.sqrt(jnp.sum(t * t, axis=1, keepdims=True))
        n = jnp.maximum(n, 1e-12)
        bi = t / n
        s = lax.dot_general(bi.astype(jnp.bfloat16), cache,
                            (((1,), (1,)), ((), ())),
                            preferred_element_type=jnp.float32,
                            precision=_PREC)
        mm = s.shape[1]
        rmax = jnp.max(s, axis=1, keepdims=True)
        jidx = lax.broadcasted_iota(jnp.int32, s.shape, 1)
        rarg = jnp.min(jnp.where(s == rmax, jidx, mm), axis=1)
        swrows_ref[pl.ds(hh * hb, hb), :] = jnp.exp(rmax) * bi
        rarg_ref[0, pl.ds(i * nb + hh * hb, hb)] = rarg
        pm = jnp.max(s, axis=0, keepdims=True)
        ct = jnp.sum(jnp.where(rarg[:, None] == jidx, 1.0, 0.0),
                     axis=0, keepdims=True)
        pcmax = pm if pcmax is None else jnp.maximum(pcmax, pm)
        cnt = ct if cnt is None else cnt + ct

    @pl.when(i == 0)
    def _():
        cmax_ref[...] = pcmax
        wsum_ref[...] = cnt

    @pl.when(i > 0)
    def _():
        cmax_ref[...] = jnp.maximum(cmax_ref[...], pcmax)
        wsum_ref[...] += cnt

    @pl.when(i == nsteps - 1)
    def _():
        ecol_ref[...] = jnp.exp(-cmax_ref[...])


def _sc_scatter_body(swrows_hbm, rarg_hbm, out_hbm,
                     rows, zbuf, idx_a, idx_b, shared,
                     *, m, q, tpt):
    c = lax.axis_index("c")
    sid = lax.axis_index("s")
    base = sid * tpt
    zero16 = jnp.zeros((16,), jnp.float32)

    pltpu.sync_copy(rarg_hbm.at[0, pl.ds(base, 128)], idx_a)
    pltpu.sync_copy(rarg_hbm.at[0, pl.ds(base + 128, 128)], idx_b)

    def zbody(t, carry):
        for r in range(q // 16):
            zbuf[t, pl.ds(r * 16, 16)] = zero16
        return carry

    lax.fori_loop(0, tpt, zbody, 0)

    for h in range(3):
        qi = c * 3 + h

        pltpu.sync_copy(zbuf, shared.at[pl.ds(base, tpt)])

        pltpu.sync_copy(swrows_hbm.at[pl.ds(base, tpt), pl.ds(qi * q, q)],
                        rows)

        plsc.subcore_barrier()

        pltpu.sync_copy(rows.at[pl.ds(0, 128)], shared.at[idx_a], add=True)
        pltpu.sync_copy(rows.at[pl.ds(128, 128)], shared.at[idx_b], add=True)
        plsc.subcore_barrier()

        pltpu.sync_copy(shared.at[pl.ds(base, tpt)],
                        out_hbm.at[qi, pl.ds(base, tpt)])


def _final_kernel(cache_ref, c0_ref, c1_ref, c2_ref, c3_ref, c4_ref, c5_ref,
                  ecol_ref, wsum_ref, upd_ref, *, mb, q):
    i = pl.program_id(0)
    c = cache_ref[...]
    ws = wsum_ref[0, pl.ds(i * mb, mb)]
    ec = ecol_ref[0, pl.ds(i * mb, mb)]
    upd = ws[:, None] > 0
    scale = (1.0 - MOM) * ec[:, None]
    quarters = (c0_ref, c1_ref, c2_ref, c3_ref, c4_ref, c5_ref)
    blended = []
    n2 = jnp.zeros((c.shape[0], 1), jnp.float32)
    for k, qref in enumerate(quarters):
        b = jnp.where(upd, MOM * c[:, k * q:(k + 1) * q]
                      + scale * qref[0], c[:, k * q:(k + 1) * q])
        blended.append(b)
        n2 = n2 + jnp.sum(b * b, axis=1, keepdims=True)
    inv = 1.0 / jnp.maximum(jnp.sqrt(n2), 1e-12)
    for k, b in enumerate(blended):
        upd_ref[:, k * q:(k + 1) * q] = b * inv


def kernel(text_token, image_token, W, cache):
    C, D = text_token.shape
    N = image_token.shape[0]
    M = cache.shape[0]
    CB = 512
    NB = 512
    MB = 512
    Q = D // 6
    TPT = N // 16
    W1 = W[:, :D].astype(jnp.bfloat16)
    W2 = W[:, D:].astype(jnp.bfloat16)
    cache_bf = cache.astype(jnp.bfloat16)

    nsteps = N // NB
    swrows, rarg, ecol, wsum = pl.pallas_call(
        functools.partial(_stats_kernel, nb=NB, nsteps=nsteps, halves=2),
        grid=(nsteps,),
        in_specs=[
            pl.BlockSpec((NB, D), lambda i: (i, 0)),
            pl.BlockSpec((M, D), lambda i: (0, 0)),
        ],
        out_specs=[
            pl.BlockSpec((NB, D), lambda i: (i, 0)),
            pl.BlockSpec((1, N), lambda i: (0, 0)),
            pl.BlockSpec((1, M), lambda i: (0, 0)),
            pl.BlockSpec((1, M), lambda i: (0, 0)),
        ],
        out_shape=[
            jax.ShapeDtypeStruct((N, D), jnp.float32),
            jax.ShapeDtypeStruct((1, N), jnp.int32),
            jax.ShapeDtypeStruct((1, M), jnp.float32),
            jax.ShapeDtypeStruct((1, M), jnp.float32),
        ],
        scratch_shapes=[
            pltpu.VMEM((1, M), jnp.float32),
        ],
    )(image_token, cache_bf)

    sc_scatter = functools.partial(
        pl.kernel,
        mesh=plsc.VectorSubcoreMesh(core_axis_name="c", subcore_axis_name="s"),
        out_type=jax.ShapeDtypeStruct((6, M, Q), jnp.float32),
        scratch_types=[
            pltpu.VMEM((TPT, Q), jnp.float32),
            pltpu.VMEM((TPT, Q), jnp.float32),
            pltpu.VMEM((128,), jnp.int32),
            pltpu.VMEM((128,), jnp.int32),
            pltpu.VMEM_SHARED((M, Q), jnp.float32),
        ],
    )(functools.partial(_sc_scatter_body, m=M, q=Q, tpt=TPT))
    contrib4 = sc_scatter(swrows, rarg)

    out, loss_sum = pl.pallas_call(
        functools.partial(_read_kernel, halves=2),
        grid=(C // CB,),
        in_specs=[
            pl.BlockSpec((CB, D), lambda i: (i, 0)),
            pl.BlockSpec((M, D), lambda i: (0, 0)),
            pl.BlockSpec((D, D), lambda i: (0, 0)),
            pl.BlockSpec((D, D), lambda i: (0, 0)),
        ],
        out_specs=[
            pl.BlockSpec((CB, D), lambda i: (i, 0)),
            pl.BlockSpec((1, 1), lambda i: (0, 0)),
        ],
        out_shape=[
            jax.ShapeDtypeStruct((C, D), jnp.float32),
            jax.ShapeDtypeStruct((1, 1), jnp.float32),
        ],
    )(text_token, cache_bf, W1, W2)
    loss = loss_sum[0, 0] / (C * D)

    updated = pl.pallas_call(
        functools.partial(_final_kernel, mb=MB, q=Q),
        grid=(M // MB,),
        in_specs=[
            pl.BlockSpec((MB, D), lambda i: (i, 0)),
            pl.BlockSpec((1, MB, Q), lambda i: (0, i, 0)),
            pl.BlockSpec((1, MB, Q), lambda i: (1, i, 0)),
            pl.BlockSpec((1, MB, Q), lambda i: (2, i, 0)),
            pl.BlockSpec((1, MB, Q), lambda i: (3, i, 0)),
            pl.BlockSpec((1, MB, Q), lambda i: (4, i, 0)),
            pl.BlockSpec((1, MB, Q), lambda i: (5, i, 0)),
            pl.BlockSpec((1, M), lambda i: (0, 0)),
            pl.BlockSpec((1, M), lambda i: (0, 0)),
        ],
        out_specs=pl.BlockSpec((MB, D), lambda i: (i, 0)),
        out_shape=jax.ShapeDtypeStruct((M, D), jnp.float32),
    )(cache, contrib4, contrib4, contrib4, contrib4, contrib4, contrib4,
      ecol, wsum)

    return (out, loss, updated)
